# baseline (speedup 1.0000x reference)
import jax
import jax.numpy as jnp
from jax.experimental import pallas as pl
from jax.experimental.pallas import tpu as pltpu


def _mse_body(thr_ref, yp_ref, yt_ref, out_ref):
    g = pl.program_id(0)
    c = yp_ref.shape[1]
    col = jax.lax.broadcasted_iota(jnp.int32, yp_ref.shape, 1) + g * c
    thr = thr_ref[:, :]
    mask = col < thr
    d = yp_ref[:, :] - yt_ref[:, :]
    part = jnp.sum(jnp.where(mask, d * d, 0.0))

    @pl.when(g == 0)
    def _init():
        out_ref[0, 0] = 0.0

    out_ref[0, 0] += part

    @pl.when(g == pl.num_programs(0) - 1)
    def _final():
        cnt = jnp.sum(thr).astype(jnp.float32)
        out_ref[0, 0] = out_ref[0, 0] / cnt


def kernel(y_pred, y_true, lengths):
    b, n = y_pred.shape[0], y_pred.shape[1]
    w = n * 16
    yp = y_pred.reshape(b, w)
    yt = y_true.reshape(b, w)
    thr = (jnp.maximum(lengths.astype(jnp.int32) - 1, 0) * 16).reshape(b, 1)

    blk = 8192
    grid = (pl.cdiv(w, blk),)
    out = pl.pallas_call(
        _mse_body,
        grid=grid,
        in_specs=[
            pl.BlockSpec((b, 1), lambda g: (0, 0)),
            pl.BlockSpec((b, blk), lambda g: (0, g)),
            pl.BlockSpec((b, blk), lambda g: (0, g)),
        ],
        out_specs=pl.BlockSpec(memory_space=pltpu.SMEM),
        out_shape=jax.ShapeDtypeStruct((1, 1), jnp.float32),
    )(thr, yp, yt)
    return out[0, 0]


# speedup vs baseline: 9.7988x; 9.7988x over previous
import jax
import jax.numpy as jnp
from jax.experimental import pallas as pl
from jax.experimental.pallas import tpu as pltpu


def _mse_body(thr_ref, yp_ref, yt_ref, out_ref):
    g = pl.program_id(0)
    c = yp_ref.shape[3]
    n = jax.lax.broadcasted_iota(jnp.int32, yp_ref.shape, 3) + g * c
    thr = thr_ref[:, :, :, :]
    mask = n < thr
    d = yp_ref[:, :, :, :] - yt_ref[:, :, :, :]
    part = jnp.sum(jnp.where(mask, d * d, 0.0))

    @pl.when(g == 0)
    def _init():
        out_ref[0, 0] = 0.0

    out_ref[0, 0] += part

    @pl.when(g == pl.num_programs(0) - 1)
    def _final():
        cnt = jnp.sum(thr).astype(jnp.float32) * 16.0
        out_ref[0, 0] = out_ref[0, 0] / cnt


def kernel(y_pred, y_true, lengths):
    b, n = y_pred.shape[0], y_pred.shape[1]
    yp = jnp.transpose(y_pred, (0, 2, 3, 1))
    yt = jnp.transpose(y_true, (0, 2, 3, 1))
    thr = jnp.maximum(lengths.astype(jnp.int32) - 1, 0).reshape(b, 1, 1, 1)

    blk = 512
    grid = (pl.cdiv(n, blk),)
    out = pl.pallas_call(
        _mse_body,
        grid=grid,
        in_specs=[
            pl.BlockSpec((b, 1, 1, 1), lambda g: (0, 0, 0, 0)),
            pl.BlockSpec((b, 4, 4, blk), lambda g: (0, 0, 0, g)),
            pl.BlockSpec((b, 4, 4, blk), lambda g: (0, 0, 0, g)),
        ],
        out_specs=pl.BlockSpec(memory_space=pltpu.SMEM),
        out_shape=jax.ShapeDtypeStruct((1, 1), jnp.float32),
    )(thr, yp, yt)
    return out[0, 0]
